# Initial kernel scaffold; baseline (speedup 1.0000x reference)
import jax
import jax.numpy as jnp
from jax.experimental import pallas as pl


def kernel(F_E, Q_prime, W1, b1, g1, bt1, W2, b2, g2, bt2, Ww, bw, gw, btw):
    raise NotImplementedError("write your pallas kernel here")


# speedup vs baseline: 10.5068x; 10.5068x over previous
import functools

import jax
import jax.numpy as jnp
from jax import lax
from jax.experimental import pallas as pl
from jax.experimental.pallas import tpu as pltpu
from jax.experimental.pallas import tpu_sc as plsc

KNB = 16
B, C, RN = 8, 128, 2048
NPTS = B * RN
NROWS = NPTS * KNB
EPS = 1e-5
HIGH = lax.Precision.HIGHEST
RB = 256
NSTEP = NPTS // RB
KROW = 256


def _tables_body(fe_ref, q_ref, w2f_ref, b2_ref, w1_ref, ww_ref,
                 g_ref, pq_ref):
    fe = fe_ref[0]
    q = q_ref[0]
    g = lax.dot_general(fe, w2f_ref[...], (((0,), (1,)), ((), ())),
                        precision=HIGH)
    g_ref[0] = g + b2_ref[...]
    p = lax.dot_general(q, w1_ref[...], (((0,), (1,)), ((), ())),
                        precision=HIGH)
    pw = lax.dot_general(q, ww_ref[...], (((0,), (1,)), ((), ())),
                         precision=HIGH)
    pad = jnp.zeros((RN, 48), jnp.float32)
    pq_ref[0] = jnp.concatenate([p, pw, pad], axis=1)


def _make_tables(F_E, Q_prime, W2F, b2, W1, Ww):
    return pl.pallas_call(
        _tables_body,
        grid=(B,),
        in_specs=[
            pl.BlockSpec((1, C, RN), lambda b: (b, 0, 0)),
            pl.BlockSpec((1, 3, RN), lambda b: (b, 0, 0)),
            pl.BlockSpec((C, C), lambda b: (0, 0)),
            pl.BlockSpec((1, C), lambda b: (0, 0)),
            pl.BlockSpec((64, 3), lambda b: (0, 0)),
            pl.BlockSpec((KNB, 3), lambda b: (0, 0)),
        ],
        out_specs=[
            pl.BlockSpec((1, RN, C), lambda b: (b, 0, 0)),
            pl.BlockSpec((1, RN, C), lambda b: (b, 0, 0)),
        ],
        out_shape=[
            jax.ShapeDtypeStruct((B, RN, C), jnp.float32),
            jax.ShapeDtypeStruct((B, RN, C), jnp.float32),
        ],
    )(F_E, Q_prime, W2F, b2, W1, Ww)


def _knn_body(q_ref, idx_ref):
    b = pl.program_id(0)
    j = pl.program_id(1)
    q = q_ref[0]
    sq = jnp.sum(q * q, axis=0, keepdims=True)
    qb = q_ref[0, :, pl.ds(j * KROW, KROW)]
    scores = lax.dot_general(qb, q, (((0,), (0,)), ((), ())))
    sqb = jnp.reshape(jnp.sum(qb * qb, axis=0), (KROW, 1))
    d = (sqb - 2.0 * scores) + sq
    iota = lax.broadcasted_iota(jnp.int32, (KROW, RN), 1)
    cols = []
    for _ in range(KNB):
        m = jnp.min(d, axis=1, keepdims=True)
        sel = jnp.where(d == m, iota, RN)
        it = jnp.min(sel, axis=1)
        d = jnp.where(iota == it[:, None], jnp.inf, d)
        cols.append(jnp.reshape(it + b * RN, (KROW, 1)))
    idx_ref[0] = jnp.concatenate(cols, axis=1)


def _knn(Q_prime):
    return pl.pallas_call(
        _knn_body,
        grid=(B, RN // KROW),
        in_specs=[pl.BlockSpec((1, 3, RN), lambda b, j: (b, 0, 0))],
        out_specs=pl.BlockSpec((1, KROW, KNB), lambda b, j: (b, j, 0)),
        out_shape=jax.ShapeDtypeStruct((B, RN, KNB), jnp.int32),
    )(Q_prime)


def _sc_gather(tg, tpq, idx2d):
    NC, NS = 2, 16
    NW = NC * NS
    rows_per_w = idx2d.shape[0] // NW

    @functools.partial(
        pl.kernel,
        mesh=plsc.VectorSubcoreMesh(core_axis_name="c", subcore_axis_name="s"),
        out_type=(
            jax.ShapeDtypeStruct((NROWS, C), jnp.float32),
            jax.ShapeDtypeStruct((NROWS, C), jnp.float32),
        ),
        scratch_types=[
            pltpu.VMEM((128,), jnp.int32),
            pltpu.VMEM((128, C), jnp.float32),
            pltpu.VMEM((128, C), jnp.float32),
            pltpu.SemaphoreType.DMA,
        ],
    )
    def gk(tg_h, tpq_h, idx_h, og_h, opq_h, idx_v, bg, bpq, sem):
        wid = lax.axis_index("s") * NC + lax.axis_index("c")

        def body(c, carry):
            row = wid * rows_per_w + c
            pltpu.sync_copy(idx_h.at[row], idx_v)
            c1 = pltpu.async_copy(tg_h.at[idx_v], bg, sem)
            c2 = pltpu.async_copy(tpq_h.at[idx_v], bpq, sem)
            c1.wait()
            c2.wait()
            out0 = row * 128
            pltpu.sync_copy(bg, og_h.at[pl.ds(out0, 128)])
            pltpu.sync_copy(bpq, opq_h.at[pl.ds(out0, 128)])
            return carry

        lax.fori_loop(0, rows_per_w, body, 0)

    return gk(tg, tpq, idx2d)


def _stats1_body(apq_ref, pqc_ref, b1_ref, bw_ref,
                 s1_ref, s1q_ref, sw_ref, swq_ref):
    @pl.when(pl.program_id(0) == 0)
    def _init():
        s1_ref[...] = jnp.zeros_like(s1_ref)
        s1q_ref[...] = jnp.zeros_like(s1q_ref)
        sw_ref[...] = jnp.zeros_like(sw_ref)
        swq_ref[...] = jnp.zeros_like(swq_ref)

    diff = apq_ref[...] - pqc_ref[...]
    y1 = diff[:, :, 0:64] + b1_ref[...]
    yw = diff[:, :, 64:80] + bw_ref[...]
    s1_ref[...] += jnp.sum(jnp.sum(y1, axis=0), axis=0, keepdims=True)
    s1q_ref[...] += jnp.sum(jnp.sum(y1 * y1, axis=0), axis=0, keepdims=True)
    sw_ref[...] += jnp.sum(jnp.sum(yw, axis=0), axis=0, keepdims=True)
    swq_ref[...] += jnp.sum(jnp.sum(yw * yw, axis=0), axis=0, keepdims=True)


def _stats1(APQ, PQc, b1r, bwr):
    return pl.pallas_call(
        _stats1_body,
        grid=(NSTEP,),
        in_specs=[
            pl.BlockSpec((RB, KNB, C), lambda i: (i, 0, 0)),
            pl.BlockSpec((RB, 1, C), lambda i: (i, 0, 0)),
            pl.BlockSpec((1, 1, 64), lambda i: (0, 0, 0)),
            pl.BlockSpec((1, 1, KNB), lambda i: (0, 0, 0)),
        ],
        out_specs=[
            pl.BlockSpec((1, 64), lambda i: (0, 0)),
            pl.BlockSpec((1, 64), lambda i: (0, 0)),
            pl.BlockSpec((1, KNB), lambda i: (0, 0)),
            pl.BlockSpec((1, KNB), lambda i: (0, 0)),
        ],
        out_shape=[
            jax.ShapeDtypeStruct((1, 64), jnp.float32),
            jax.ShapeDtypeStruct((1, 64), jnp.float32),
            jax.ShapeDtypeStruct((1, KNB), jnp.float32),
            jax.ShapeDtypeStruct((1, KNB), jnp.float32),
        ],
    )(APQ, PQc, b1r, bwr)


def _bn1_consts(s1, s1q, g1_ref, bt1_ref):
    n = jnp.float32(NROWS)
    m1 = s1 / n
    v1 = s1q / n - m1 * m1
    sc1 = g1_ref[...] * lax.rsqrt(v1 + EPS)
    sh1 = bt1_ref[...] - m1 * sc1
    return sc1, sh1


def _stats2_body(ag_ref, apq_ref, pqc_ref, b1_ref, g1_ref, bt1_ref,
                 s1_ref, s1q_ref, w2e_ref, s2_ref, s2q_ref):
    @pl.when(pl.program_id(0) == 0)
    def _init():
        s2_ref[...] = jnp.zeros_like(s2_ref)
        s2q_ref[...] = jnp.zeros_like(s2q_ref)

    sc1, sh1 = _bn1_consts(s1_ref[...][None], s1q_ref[...][None],
                           g1_ref, bt1_ref)
    diff = apq_ref[...] - pqc_ref[...]
    y1 = diff[:, :, 0:64] + b1_ref[...]
    enc1 = jnp.maximum(y1 * sc1 + sh1, 0.0)
    y2 = ag_ref[...] + lax.dot_general(
        enc1, w2e_ref[...], (((2,), (1,)), ((), ())), precision=HIGH)
    s2_ref[...] += jnp.sum(jnp.sum(y2, axis=0), axis=0, keepdims=True)
    s2q_ref[...] += jnp.sum(jnp.sum(y2 * y2, axis=0), axis=0, keepdims=True)


def _stats2(AG, APQ, PQc, b1r, g1r, bt1r, s1, s1q, W2E):
    return pl.pallas_call(
        _stats2_body,
        grid=(NSTEP,),
        in_specs=[
            pl.BlockSpec((RB, KNB, C), lambda i: (i, 0, 0)),
            pl.BlockSpec((RB, KNB, C), lambda i: (i, 0, 0)),
            pl.BlockSpec((RB, 1, C), lambda i: (i, 0, 0)),
            pl.BlockSpec((1, 1, 64), lambda i: (0, 0, 0)),
            pl.BlockSpec((1, 1, 64), lambda i: (0, 0, 0)),
            pl.BlockSpec((1, 1, 64), lambda i: (0, 0, 0)),
            pl.BlockSpec((1, 64), lambda i: (0, 0)),
            pl.BlockSpec((1, 64), lambda i: (0, 0)),
            pl.BlockSpec((C, 64), lambda i: (0, 0)),
        ],
        out_specs=[
            pl.BlockSpec((1, C), lambda i: (0, 0)),
            pl.BlockSpec((1, C), lambda i: (0, 0)),
        ],
        out_shape=[
            jax.ShapeDtypeStruct((1, C), jnp.float32),
            jax.ShapeDtypeStruct((1, C), jnp.float32),
        ],
    )(AG, APQ, PQc, b1r, g1r, bt1r, s1, s1q, W2E)


def _final_body(ag_ref, apq_ref, pqc_ref, fe_ref,
                b1_ref, g1_ref, bt1_ref, bw_ref, gw_ref, btw_ref,
                g2_ref, bt2_ref, s1_ref, s1q_ref, sw_ref, swq_ref,
                s2_ref, s2q_ref, w2e_ref, out_ref):
    n = jnp.float32(NROWS)
    sc1, sh1 = _bn1_consts(s1_ref[...][None], s1q_ref[...][None],
                           g1_ref, bt1_ref)
    diff = apq_ref[...] - pqc_ref[...]
    y1 = diff[:, :, 0:64] + b1_ref[...]
    enc1 = jnp.maximum(y1 * sc1 + sh1, 0.0)
    y2 = ag_ref[...] + lax.dot_general(
        enc1, w2e_ref[...], (((2,), (1,)), ((), ())), precision=HIGH)
    m2 = s2_ref[...] / n
    v2 = s2q_ref[...] / n - m2 * m2
    sc2 = g2_ref[...] * lax.rsqrt(v2 + EPS)
    sh2 = bt2_ref[...] - m2 * sc2
    enc2 = jnp.maximum(y2 * sc2[None] + sh2[None], 0.0)
    yw = diff[:, :, 64:80] + bw_ref[...]
    eye = (lax.broadcasted_iota(jnp.int32, (KNB, KNB), 0)
           == lax.broadcasted_iota(jnp.int32, (KNB, KNB), 1))
    ywd = jnp.sum(yw * jnp.where(eye, 1.0, 0.0)[None], axis=2)
    mw = sw_ref[...] / n
    vw = swq_ref[...] / n - mw * mw
    scw = gw_ref[...] * lax.rsqrt(vw + EPS)
    shw = btw_ref[...] - mw * scw
    sw = jnp.maximum(ywd * scw + shw, 0.0)
    weighted = jnp.sum(enc2 * sw[:, :, None], axis=1)
    out_ref[...] = weighted + fe_ref[...]


def _final(AG, APQ, PQc, Fe2d, b1r, g1r, bt1r, bwr, gwr, btwr,
           g2r, bt2r, s1, s1q, sw_, swq, s2, s2q, W2E):
    small64 = pl.BlockSpec((1, 1, 64), lambda i: (0, 0, 0))
    small16 = pl.BlockSpec((1, 1, KNB), lambda i: (0, 0, 0))
    vec64 = pl.BlockSpec((1, 64), lambda i: (0, 0))
    vec16 = pl.BlockSpec((1, KNB), lambda i: (0, 0))
    vecC = pl.BlockSpec((1, C), lambda i: (0, 0))
    return pl.pallas_call(
        _final_body,
        grid=(NSTEP,),
        in_specs=[
            pl.BlockSpec((RB, KNB, C), lambda i: (i, 0, 0)),
            pl.BlockSpec((RB, KNB, C), lambda i: (i, 0, 0)),
            pl.BlockSpec((RB, 1, C), lambda i: (i, 0, 0)),
            pl.BlockSpec((RB, C), lambda i: (i, 0)),
            small64, small64, small64,
            small16, vec16, vec16,
            vecC, vecC,
            vec64, vec64, vec16, vec16, vecC, vecC,
            pl.BlockSpec((C, 64), lambda i: (0, 0)),
        ],
        out_specs=pl.BlockSpec((RB, C), lambda i: (i, 0)),
        out_shape=jax.ShapeDtypeStruct((NPTS, C), jnp.float32),
    )(AG, APQ, PQc, Fe2d, b1r, g1r, bt1r, bwr, gwr, btwr,
      g2r, bt2r, s1, s1q, sw_, swq, s2, s2q, W2E)


def kernel(F_E, Q_prime, W1, b1, g1, bt1, W2, b2, g2, bt2, Ww, bw, gw, btw):
    W2F = W2[:, :C]
    W2E = W2[:, C:]
    G, PQ = _make_tables(F_E, Q_prime, W2F, b2.reshape(1, C), W1, Ww)
    idx = _knn(Q_prime)
    AG, APQ = _sc_gather(G.reshape(NPTS, C), PQ.reshape(NPTS, C),
                         idx.reshape(NROWS // 128, 128))
    AG = AG.reshape(NPTS, KNB, C)
    APQ = APQ.reshape(NPTS, KNB, C)
    PQc = PQ.reshape(NPTS, 1, C)
    b1r = b1.reshape(1, 1, 64)
    bwr = bw.reshape(1, 1, KNB)
    s1, s1q, sw_, swq = _stats1(APQ, PQc, b1r, bwr)
    g1r = g1.reshape(1, 1, 64)
    bt1r = bt1.reshape(1, 1, 64)
    s2, s2q = _stats2(AG, APQ, PQc, b1r, g1r, bt1r, s1, s1q, W2E)
    Fe2d = jnp.transpose(F_E, (0, 2, 1)).reshape(NPTS, C)
    out2d = _final(AG, APQ, PQc, Fe2d, b1r, g1r, bt1r,
                   bwr, gw.reshape(1, KNB), btw.reshape(1, KNB),
                   g2.reshape(1, C), bt2.reshape(1, C),
                   s1, s1q, sw_, swq, s2, s2q, W2E)
    return jnp.transpose(out2d.reshape(B, RN, C), (0, 2, 1))


# speedup vs baseline: 11.8681x; 1.1296x over previous
import functools

import jax
import jax.numpy as jnp
from jax import lax
from jax.experimental import pallas as pl
from jax.experimental.pallas import tpu as pltpu
from jax.experimental.pallas import tpu_sc as plsc

KNB = 16
B, C, RN = 8, 128, 2048
NPTS = B * RN
NROWS = NPTS * KNB
EPS = 1e-5
HIGH = lax.Precision.HIGHEST
RB = 256
NSTEP = NPTS // RB
KROW = 256


def _tables_body(fe_ref, q_ref, w2f_ref, b2_ref, w1_ref, ww_ref,
                 g_ref, pq_ref):
    fe = fe_ref[0]
    q = q_ref[0]
    g = lax.dot_general(fe, w2f_ref[...], (((0,), (1,)), ((), ())),
                        precision=HIGH)
    g_ref[0] = g + b2_ref[...]
    p = lax.dot_general(q, w1_ref[...], (((0,), (1,)), ((), ())),
                        precision=HIGH)
    pw = lax.dot_general(q, ww_ref[...], (((0,), (1,)), ((), ())),
                         precision=HIGH)
    pad = jnp.zeros((RN, 48), jnp.float32)
    pq_ref[0] = jnp.concatenate([p, pw, pad], axis=1)


def _make_tables(F_E, Q_prime, W2F, b2, W1, Ww):
    return pl.pallas_call(
        _tables_body,
        grid=(B,),
        in_specs=[
            pl.BlockSpec((1, C, RN), lambda b: (b, 0, 0)),
            pl.BlockSpec((1, 3, RN), lambda b: (b, 0, 0)),
            pl.BlockSpec((C, C), lambda b: (0, 0)),
            pl.BlockSpec((1, C), lambda b: (0, 0)),
            pl.BlockSpec((64, 3), lambda b: (0, 0)),
            pl.BlockSpec((KNB, 3), lambda b: (0, 0)),
        ],
        out_specs=[
            pl.BlockSpec((1, RN, C), lambda b: (b, 0, 0)),
            pl.BlockSpec((1, RN, C), lambda b: (b, 0, 0)),
        ],
        out_shape=[
            jax.ShapeDtypeStruct((B, RN, C), jnp.float32),
            jax.ShapeDtypeStruct((B, RN, C), jnp.float32),
        ],
    )(F_E, Q_prime, W2F, b2, W1, Ww)


def _knn_body(q_ref, idx_ref):
    b = pl.program_id(0)
    j = pl.program_id(1)
    q = q_ref[0]
    sq = jnp.sum(q * q, axis=0, keepdims=True)
    qb = q_ref[0, :, pl.ds(j * KROW, KROW)]
    scores = lax.dot_general(qb, q, (((0,), (0,)), ((), ())))
    sqb = jnp.reshape(jnp.sum(qb * qb, axis=0), (KROW, 1))
    d = (sqb - 2.0 * scores) + sq
    iota = lax.broadcasted_iota(jnp.int32, (KROW, RN), 1).astype(jnp.float32)
    big = jnp.float32(RN)
    cols = []
    for _ in range(KNB):
        m = jnp.min(d, axis=1, keepdims=True)
        sel = jnp.where(d == m, iota, big)
        it = jnp.min(sel, axis=1, keepdims=True)
        d = jnp.where(iota == it, jnp.inf, d)
        cols.append(it.astype(jnp.int32) + b * RN)
    idx_ref[0] = jnp.concatenate(cols, axis=1)


def _knn(Q_prime):
    return pl.pallas_call(
        _knn_body,
        grid=(B, RN // KROW),
        in_specs=[pl.BlockSpec((1, 3, RN), lambda b, j: (b, 0, 0))],
        out_specs=pl.BlockSpec((1, KROW, KNB), lambda b, j: (b, j, 0)),
        out_shape=jax.ShapeDtypeStruct((B, RN, KNB), jnp.int32),
    )(Q_prime)


def _sc_gather(tg, tpq, idx2d):
    NC, NS = 2, 16
    NW = NC * NS
    rows_per_w = idx2d.shape[0] // NW

    @functools.partial(
        pl.kernel,
        mesh=plsc.VectorSubcoreMesh(core_axis_name="c", subcore_axis_name="s"),
        out_type=(
            jax.ShapeDtypeStruct((NROWS, C), jnp.float32),
            jax.ShapeDtypeStruct((NROWS, C), jnp.float32),
        ),
        scratch_types=[
            pltpu.VMEM((128,), jnp.int32),
            pltpu.VMEM((128, C), jnp.float32),
            pltpu.VMEM((128, C), jnp.float32),
            pltpu.SemaphoreType.DMA,
        ],
    )
    def gk(tg_h, tpq_h, idx_h, og_h, opq_h, idx_v, bg, bpq, sem):
        wid = lax.axis_index("s") * NC + lax.axis_index("c")

        def body(c, carry):
            row = wid * rows_per_w + c
            pltpu.sync_copy(idx_h.at[row], idx_v)
            c1 = pltpu.async_copy(tg_h.at[idx_v], bg, sem)
            c2 = pltpu.async_copy(tpq_h.at[idx_v], bpq, sem)
            c1.wait()
            c2.wait()
            out0 = row * 128
            pltpu.sync_copy(bg, og_h.at[pl.ds(out0, 128)])
            pltpu.sync_copy(bpq, opq_h.at[pl.ds(out0, 128)])
            return carry

        lax.fori_loop(0, rows_per_w, body, 0)

    return gk(tg, tpq, idx2d)


def _stats1_body(apq_ref, pqc_ref, b1_ref, bw_ref,
                 s1_ref, s1q_ref, sw_ref, swq_ref):
    @pl.when(pl.program_id(0) == 0)
    def _init():
        s1_ref[...] = jnp.zeros_like(s1_ref)
        s1q_ref[...] = jnp.zeros_like(s1q_ref)
        sw_ref[...] = jnp.zeros_like(sw_ref)
        swq_ref[...] = jnp.zeros_like(swq_ref)

    diff = apq_ref[...] - pqc_ref[...]
    y1 = diff[:, :, 0:64] + b1_ref[...]
    yw = diff[:, :, 64:80] + bw_ref[...]
    s1_ref[...] += jnp.sum(jnp.sum(y1, axis=0), axis=0, keepdims=True)
    s1q_ref[...] += jnp.sum(jnp.sum(y1 * y1, axis=0), axis=0, keepdims=True)
    sw_ref[...] += jnp.sum(jnp.sum(yw, axis=0), axis=0, keepdims=True)
    swq_ref[...] += jnp.sum(jnp.sum(yw * yw, axis=0), axis=0, keepdims=True)


def _stats1(APQ, PQc, b1r, bwr):
    return pl.pallas_call(
        _stats1_body,
        grid=(NSTEP,),
        in_specs=[
            pl.BlockSpec((RB, KNB, C), lambda i: (i, 0, 0)),
            pl.BlockSpec((RB, 1, C), lambda i: (i, 0, 0)),
            pl.BlockSpec((1, 1, 64), lambda i: (0, 0, 0)),
            pl.BlockSpec((1, 1, KNB), lambda i: (0, 0, 0)),
        ],
        out_specs=[
            pl.BlockSpec((1, 64), lambda i: (0, 0)),
            pl.BlockSpec((1, 64), lambda i: (0, 0)),
            pl.BlockSpec((1, KNB), lambda i: (0, 0)),
            pl.BlockSpec((1, KNB), lambda i: (0, 0)),
        ],
        out_shape=[
            jax.ShapeDtypeStruct((1, 64), jnp.float32),
            jax.ShapeDtypeStruct((1, 64), jnp.float32),
            jax.ShapeDtypeStruct((1, KNB), jnp.float32),
            jax.ShapeDtypeStruct((1, KNB), jnp.float32),
        ],
    )(APQ, PQc, b1r, bwr)


def _bn1_consts(s1, s1q, g1_ref, bt1_ref):
    n = jnp.float32(NROWS)
    m1 = s1 / n
    v1 = s1q / n - m1 * m1
    sc1 = g1_ref[...] * lax.rsqrt(v1 + EPS)
    sh1 = bt1_ref[...] - m1 * sc1
    return sc1, sh1


def _stats2_body(ag_ref, apq_ref, pqc_ref, b1_ref, g1_ref, bt1_ref,
                 s1_ref, s1q_ref, w2e_ref, s2_ref, s2q_ref):
    @pl.when(pl.program_id(0) == 0)
    def _init():
        s2_ref[...] = jnp.zeros_like(s2_ref)
        s2q_ref[...] = jnp.zeros_like(s2q_ref)

    sc1, sh1 = _bn1_consts(s1_ref[...][None], s1q_ref[...][None],
                           g1_ref, bt1_ref)
    diff = apq_ref[...] - pqc_ref[...]
    y1 = diff[:, :, 0:64] + b1_ref[...]
    enc1 = jnp.maximum(y1 * sc1 + sh1, 0.0)
    y2 = ag_ref[...] + lax.dot_general(
        enc1, w2e_ref[...], (((2,), (1,)), ((), ())), precision=HIGH)
    s2_ref[...] += jnp.sum(jnp.sum(y2, axis=0), axis=0, keepdims=True)
    s2q_ref[...] += jnp.sum(jnp.sum(y2 * y2, axis=0), axis=0, keepdims=True)


def _stats2(AG, APQ, PQc, b1r, g1r, bt1r, s1, s1q, W2E):
    return pl.pallas_call(
        _stats2_body,
        grid=(NSTEP,),
        in_specs=[
            pl.BlockSpec((RB, KNB, C), lambda i: (i, 0, 0)),
            pl.BlockSpec((RB, KNB, C), lambda i: (i, 0, 0)),
            pl.BlockSpec((RB, 1, C), lambda i: (i, 0, 0)),
            pl.BlockSpec((1, 1, 64), lambda i: (0, 0, 0)),
            pl.BlockSpec((1, 1, 64), lambda i: (0, 0, 0)),
            pl.BlockSpec((1, 1, 64), lambda i: (0, 0, 0)),
            pl.BlockSpec((1, 64), lambda i: (0, 0)),
            pl.BlockSpec((1, 64), lambda i: (0, 0)),
            pl.BlockSpec((C, 64), lambda i: (0, 0)),
        ],
        out_specs=[
            pl.BlockSpec((1, C), lambda i: (0, 0)),
            pl.BlockSpec((1, C), lambda i: (0, 0)),
        ],
        out_shape=[
            jax.ShapeDtypeStruct((1, C), jnp.float32),
            jax.ShapeDtypeStruct((1, C), jnp.float32),
        ],
    )(AG, APQ, PQc, b1r, g1r, bt1r, s1, s1q, W2E)


def _final_body(ag_ref, apq_ref, pqc_ref, fe_ref,
                b1_ref, g1_ref, bt1_ref, bw_ref, gw_ref, btw_ref,
                g2_ref, bt2_ref, s1_ref, s1q_ref, sw_ref, swq_ref,
                s2_ref, s2q_ref, w2e_ref, out_ref):
    n = jnp.float32(NROWS)
    sc1, sh1 = _bn1_consts(s1_ref[...][None], s1q_ref[...][None],
                           g1_ref, bt1_ref)
    diff = apq_ref[...] - pqc_ref[...]
    y1 = diff[:, :, 0:64] + b1_ref[...]
    enc1 = jnp.maximum(y1 * sc1 + sh1, 0.0)
    y2 = ag_ref[...] + lax.dot_general(
        enc1, w2e_ref[...], (((2,), (1,)), ((), ())), precision=HIGH)
    m2 = s2_ref[...] / n
    v2 = s2q_ref[...] / n - m2 * m2
    sc2 = g2_ref[...] * lax.rsqrt(v2 + EPS)
    sh2 = bt2_ref[...] - m2 * sc2
    enc2 = jnp.maximum(y2 * sc2[None] + sh2[None], 0.0)
    yw = diff[:, :, 64:80] + bw_ref[...]
    eye = (lax.broadcasted_iota(jnp.int32, (KNB, KNB), 0)
           == lax.broadcasted_iota(jnp.int32, (KNB, KNB), 1))
    ywd = jnp.sum(yw * jnp.where(eye, 1.0, 0.0)[None], axis=2)
    mw = sw_ref[...] / n
    vw = swq_ref[...] / n - mw * mw
    scw = gw_ref[...] * lax.rsqrt(vw + EPS)
    shw = btw_ref[...] - mw * scw
    sw = jnp.maximum(ywd * scw + shw, 0.0)
    weighted = jnp.sum(enc2 * sw[:, :, None], axis=1)
    out_ref[...] = weighted + fe_ref[...]


def _final(AG, APQ, PQc, Fe2d, b1r, g1r, bt1r, bwr, gwr, btwr,
           g2r, bt2r, s1, s1q, sw_, swq, s2, s2q, W2E):
    small64 = pl.BlockSpec((1, 1, 64), lambda i: (0, 0, 0))
    small16 = pl.BlockSpec((1, 1, KNB), lambda i: (0, 0, 0))
    vec64 = pl.BlockSpec((1, 64), lambda i: (0, 0))
    vec16 = pl.BlockSpec((1, KNB), lambda i: (0, 0))
    vecC = pl.BlockSpec((1, C), lambda i: (0, 0))
    return pl.pallas_call(
        _final_body,
        grid=(NSTEP,),
        in_specs=[
            pl.BlockSpec((RB, KNB, C), lambda i: (i, 0, 0)),
            pl.BlockSpec((RB, KNB, C), lambda i: (i, 0, 0)),
            pl.BlockSpec((RB, 1, C), lambda i: (i, 0, 0)),
            pl.BlockSpec((RB, C), lambda i: (i, 0)),
            small64, small64, small64,
            small16, vec16, vec16,
            vecC, vecC,
            vec64, vec64, vec16, vec16, vecC, vecC,
            pl.BlockSpec((C, 64), lambda i: (0, 0)),
        ],
        out_specs=pl.BlockSpec((RB, C), lambda i: (i, 0)),
        out_shape=jax.ShapeDtypeStruct((NPTS, C), jnp.float32),
    )(AG, APQ, PQc, Fe2d, b1r, g1r, bt1r, bwr, gwr, btwr,
      g2r, bt2r, s1, s1q, sw_, swq, s2, s2q, W2E)


def kernel(F_E, Q_prime, W1, b1, g1, bt1, W2, b2, g2, bt2, Ww, bw, gw, btw):
    W2F = W2[:, :C]
    W2E = W2[:, C:]
    G, PQ = _make_tables(F_E, Q_prime, W2F, b2.reshape(1, C), W1, Ww)
    idx = _knn(Q_prime)
    AG, APQ = _sc_gather(G.reshape(NPTS, C), PQ.reshape(NPTS, C),
                         idx.reshape(NROWS // 128, 128))
    AG = AG.reshape(NPTS, KNB, C)
    APQ = APQ.reshape(NPTS, KNB, C)
    PQc = PQ.reshape(NPTS, 1, C)
    b1r = b1.reshape(1, 1, 64)
    bwr = bw.reshape(1, 1, KNB)
    s1, s1q, sw_, swq = _stats1(APQ, PQc, b1r, bwr)
    g1r = g1.reshape(1, 1, 64)
    bt1r = bt1.reshape(1, 1, 64)
    s2, s2q = _stats2(AG, APQ, PQc, b1r, g1r, bt1r, s1, s1q, W2E)
    Fe2d = jnp.transpose(F_E, (0, 2, 1)).reshape(NPTS, C)
    out2d = _final(AG, APQ, PQc, Fe2d, b1r, g1r, bt1r,
                   bwr, gw.reshape(1, KNB), btw.reshape(1, KNB),
                   g2.reshape(1, C), bt2.reshape(1, C),
                   s1, s1q, sw_, swq, s2, s2q, W2E)
    return jnp.transpose(out2d.reshape(B, RN, C), (0, 2, 1))


# speedup vs baseline: 11.9458x; 1.0066x over previous
import functools

import jax
import jax.numpy as jnp
from jax import lax
from jax.experimental import pallas as pl
from jax.experimental.pallas import tpu as pltpu
from jax.experimental.pallas import tpu_sc as plsc

KNB = 16
B, C, RN = 8, 128, 2048
NPTS = B * RN
NROWS = NPTS * KNB
EPS = 1e-5
HIGH = lax.Precision.HIGHEST
RB = 256
NSTEP = NPTS // RB
KROW = 512


def _tables_body(fe_ref, q_ref, w2f_ref, b2_ref, w1_ref, ww_ref,
                 g_ref, pq_ref):
    fe = fe_ref[0]
    q = q_ref[0]
    g = lax.dot_general(fe, w2f_ref[...], (((0,), (1,)), ((), ())),
                        precision=HIGH)
    g_ref[0] = g + b2_ref[...]
    p = lax.dot_general(q, w1_ref[...], (((0,), (1,)), ((), ())),
                        precision=HIGH)
    pw = lax.dot_general(q, ww_ref[...], (((0,), (1,)), ((), ())),
                         precision=HIGH)
    pad = jnp.zeros((RN, 48), jnp.float32)
    pq_ref[0] = jnp.concatenate([p, pw, pad], axis=1)


def _make_tables(F_E, Q_prime, W2F, b2, W1, Ww):
    return pl.pallas_call(
        _tables_body,
        grid=(B,),
        in_specs=[
            pl.BlockSpec((1, C, RN), lambda b: (b, 0, 0)),
            pl.BlockSpec((1, 3, RN), lambda b: (b, 0, 0)),
            pl.BlockSpec((C, C), lambda b: (0, 0)),
            pl.BlockSpec((1, C), lambda b: (0, 0)),
            pl.BlockSpec((64, 3), lambda b: (0, 0)),
            pl.BlockSpec((KNB, 3), lambda b: (0, 0)),
        ],
        out_specs=[
            pl.BlockSpec((1, RN, C), lambda b: (b, 0, 0)),
            pl.BlockSpec((1, RN, C), lambda b: (b, 0, 0)),
        ],
        out_shape=[
            jax.ShapeDtypeStruct((B, RN, C), jnp.float32),
            jax.ShapeDtypeStruct((B, RN, C), jnp.float32),
        ],
    )(F_E, Q_prime, W2F, b2, W1, Ww)


def _knn_body(q_ref, idx_ref):
    b = pl.program_id(0)
    j = pl.program_id(1)
    q = q_ref[0]
    sq = jnp.sum(q * q, axis=0, keepdims=True)
    qb = q_ref[0, :, pl.ds(j * KROW, KROW)]
    scores = lax.dot_general(qb, q, (((0,), (0,)), ((), ())))
    sqb = jnp.reshape(jnp.sum(qb * qb, axis=0), (KROW, 1))
    d = (sqb - 2.0 * scores) + sq
    iota = lax.broadcasted_iota(jnp.int32, (KROW, RN), 1).astype(jnp.float32)
    big = jnp.float32(RN)
    cols = []
    for _ in range(KNB):
        m = jnp.min(d, axis=1, keepdims=True)
        sel = jnp.where(d == m, iota, big)
        it = jnp.min(sel, axis=1, keepdims=True)
        d = jnp.where(iota == it, jnp.inf, d)
        cols.append(it.astype(jnp.int32) + b * RN)
    idx_ref[0] = jnp.concatenate(cols, axis=1)


def _knn(Q_prime):
    return pl.pallas_call(
        _knn_body,
        grid=(B, RN // KROW),
        in_specs=[pl.BlockSpec((1, 3, RN), lambda b, j: (b, 0, 0))],
        out_specs=pl.BlockSpec((1, KROW, KNB), lambda b, j: (b, j, 0)),
        out_shape=jax.ShapeDtypeStruct((B, RN, KNB), jnp.int32),
    )(Q_prime)


def _sc_gather(tg, tpq, idx2d):
    NC, NS = 2, 16
    NW = NC * NS
    rows_per_w = idx2d.shape[0] // NW

    @functools.partial(
        pl.kernel,
        mesh=plsc.VectorSubcoreMesh(core_axis_name="c", subcore_axis_name="s"),
        out_type=(
            jax.ShapeDtypeStruct((NROWS, C), jnp.float32),
            jax.ShapeDtypeStruct((NROWS, C), jnp.float32),
        ),
        scratch_types=[
            pltpu.VMEM((128,), jnp.int32),
            pltpu.VMEM((128, C), jnp.float32),
            pltpu.VMEM((128, C), jnp.float32),
            pltpu.SemaphoreType.DMA,
        ],
    )
    def gk(tg_h, tpq_h, idx_h, og_h, opq_h, idx_v, bg, bpq, sem):
        wid = lax.axis_index("s") * NC + lax.axis_index("c")

        def body(c, carry):
            row = wid * rows_per_w + c
            pltpu.sync_copy(idx_h.at[row], idx_v)
            c1 = pltpu.async_copy(tg_h.at[idx_v], bg, sem)
            c2 = pltpu.async_copy(tpq_h.at[idx_v], bpq, sem)
            c1.wait()
            c2.wait()
            out0 = row * 128
            pltpu.sync_copy(bg, og_h.at[pl.ds(out0, 128)])
            pltpu.sync_copy(bpq, opq_h.at[pl.ds(out0, 128)])
            return carry

        lax.fori_loop(0, rows_per_w, body, 0)

    return gk(tg, tpq, idx2d)


def _stats1_body(apq_ref, pqc_ref, b1_ref, bw_ref,
                 s1_ref, s1q_ref, sw_ref, swq_ref):
    @pl.when(pl.program_id(0) == 0)
    def _init():
        s1_ref[...] = jnp.zeros_like(s1_ref)
        s1q_ref[...] = jnp.zeros_like(s1q_ref)
        sw_ref[...] = jnp.zeros_like(sw_ref)
        swq_ref[...] = jnp.zeros_like(swq_ref)

    diff = apq_ref[...] - pqc_ref[...]
    y1 = diff[:, :, 0:64] + b1_ref[...]
    yw = diff[:, :, 64:80] + bw_ref[...]
    s1_ref[...] += jnp.sum(jnp.sum(y1, axis=0), axis=0, keepdims=True)
    s1q_ref[...] += jnp.sum(jnp.sum(y1 * y1, axis=0), axis=0, keepdims=True)
    sw_ref[...] += jnp.sum(jnp.sum(yw, axis=0), axis=0, keepdims=True)
    swq_ref[...] += jnp.sum(jnp.sum(yw * yw, axis=0), axis=0, keepdims=True)


def _stats1(APQ, PQc, b1r, bwr):
    return pl.pallas_call(
        _stats1_body,
        grid=(NSTEP,),
        in_specs=[
            pl.BlockSpec((RB, KNB, C), lambda i: (i, 0, 0)),
            pl.BlockSpec((RB, 1, C), lambda i: (i, 0, 0)),
            pl.BlockSpec((1, 1, 64), lambda i: (0, 0, 0)),
            pl.BlockSpec((1, 1, KNB), lambda i: (0, 0, 0)),
        ],
        out_specs=[
            pl.BlockSpec((1, 64), lambda i: (0, 0)),
            pl.BlockSpec((1, 64), lambda i: (0, 0)),
            pl.BlockSpec((1, KNB), lambda i: (0, 0)),
            pl.BlockSpec((1, KNB), lambda i: (0, 0)),
        ],
        out_shape=[
            jax.ShapeDtypeStruct((1, 64), jnp.float32),
            jax.ShapeDtypeStruct((1, 64), jnp.float32),
            jax.ShapeDtypeStruct((1, KNB), jnp.float32),
            jax.ShapeDtypeStruct((1, KNB), jnp.float32),
        ],
    )(APQ, PQc, b1r, bwr)


def _bn1_consts(s1, s1q, g1_ref, bt1_ref):
    n = jnp.float32(NROWS)
    m1 = s1 / n
    v1 = s1q / n - m1 * m1
    sc1 = g1_ref[...] * lax.rsqrt(v1 + EPS)
    sh1 = bt1_ref[...] - m1 * sc1
    return sc1, sh1


def _stats2_body(ag_ref, apq_ref, pqc_ref, b1_ref, g1_ref, bt1_ref,
                 s1_ref, s1q_ref, w2e_ref, s2_ref, s2q_ref):
    @pl.when(pl.program_id(0) == 0)
    def _init():
        s2_ref[...] = jnp.zeros_like(s2_ref)
        s2q_ref[...] = jnp.zeros_like(s2q_ref)

    sc1, sh1 = _bn1_consts(s1_ref[...][None], s1q_ref[...][None],
                           g1_ref, bt1_ref)
    diff = apq_ref[...] - pqc_ref[...]
    y1 = diff[:, :, 0:64] + b1_ref[...]
    enc1 = jnp.maximum(y1 * sc1 + sh1, 0.0)
    y2 = ag_ref[...] + lax.dot_general(
        enc1, w2e_ref[...], (((2,), (1,)), ((), ())), precision=HIGH)
    s2_ref[...] += jnp.sum(jnp.sum(y2, axis=0), axis=0, keepdims=True)
    s2q_ref[...] += jnp.sum(jnp.sum(y2 * y2, axis=0), axis=0, keepdims=True)


def _stats2(AG, APQ, PQc, b1r, g1r, bt1r, s1, s1q, W2E):
    return pl.pallas_call(
        _stats2_body,
        grid=(NSTEP,),
        in_specs=[
            pl.BlockSpec((RB, KNB, C), lambda i: (i, 0, 0)),
            pl.BlockSpec((RB, KNB, C), lambda i: (i, 0, 0)),
            pl.BlockSpec((RB, 1, C), lambda i: (i, 0, 0)),
            pl.BlockSpec((1, 1, 64), lambda i: (0, 0, 0)),
            pl.BlockSpec((1, 1, 64), lambda i: (0, 0, 0)),
            pl.BlockSpec((1, 1, 64), lambda i: (0, 0, 0)),
            pl.BlockSpec((1, 64), lambda i: (0, 0)),
            pl.BlockSpec((1, 64), lambda i: (0, 0)),
            pl.BlockSpec((C, 64), lambda i: (0, 0)),
        ],
        out_specs=[
            pl.BlockSpec((1, C), lambda i: (0, 0)),
            pl.BlockSpec((1, C), lambda i: (0, 0)),
        ],
        out_shape=[
            jax.ShapeDtypeStruct((1, C), jnp.float32),
            jax.ShapeDtypeStruct((1, C), jnp.float32),
        ],
    )(AG, APQ, PQc, b1r, g1r, bt1r, s1, s1q, W2E)


def _final_body(ag_ref, apq_ref, pqc_ref, fe_ref,
                b1_ref, g1_ref, bt1_ref, bw_ref, gw_ref, btw_ref,
                g2_ref, bt2_ref, s1_ref, s1q_ref, sw_ref, swq_ref,
                s2_ref, s2q_ref, w2e_ref, out_ref):
    n = jnp.float32(NROWS)
    sc1, sh1 = _bn1_consts(s1_ref[...][None], s1q_ref[...][None],
                           g1_ref, bt1_ref)
    diff = apq_ref[...] - pqc_ref[...]
    y1 = diff[:, :, 0:64] + b1_ref[...]
    enc1 = jnp.maximum(y1 * sc1 + sh1, 0.0)
    y2 = ag_ref[...] + lax.dot_general(
        enc1, w2e_ref[...], (((2,), (1,)), ((), ())), precision=HIGH)
    m2 = s2_ref[...] / n
    v2 = s2q_ref[...] / n - m2 * m2
    sc2 = g2_ref[...] * lax.rsqrt(v2 + EPS)
    sh2 = bt2_ref[...] - m2 * sc2
    enc2 = jnp.maximum(y2 * sc2[None] + sh2[None], 0.0)
    yw = diff[:, :, 64:80] + bw_ref[...]
    eye = (lax.broadcasted_iota(jnp.int32, (KNB, KNB), 0)
           == lax.broadcasted_iota(jnp.int32, (KNB, KNB), 1))
    ywd = jnp.sum(yw * jnp.where(eye, 1.0, 0.0)[None], axis=2)
    mw = sw_ref[...] / n
    vw = swq_ref[...] / n - mw * mw
    scw = gw_ref[...] * lax.rsqrt(vw + EPS)
    shw = btw_ref[...] - mw * scw
    sw = jnp.maximum(ywd * scw + shw, 0.0)
    weighted = jnp.sum(enc2 * sw[:, :, None], axis=1)
    out_ref[...] = weighted + fe_ref[...]


def _final(AG, APQ, PQc, Fe2d, b1r, g1r, bt1r, bwr, gwr, btwr,
           g2r, bt2r, s1, s1q, sw_, swq, s2, s2q, W2E):
    small64 = pl.BlockSpec((1, 1, 64), lambda i: (0, 0, 0))
    small16 = pl.BlockSpec((1, 1, KNB), lambda i: (0, 0, 0))
    vec64 = pl.BlockSpec((1, 64), lambda i: (0, 0))
    vec16 = pl.BlockSpec((1, KNB), lambda i: (0, 0))
    vecC = pl.BlockSpec((1, C), lambda i: (0, 0))
    return pl.pallas_call(
        _final_body,
        grid=(NSTEP,),
        in_specs=[
            pl.BlockSpec((RB, KNB, C), lambda i: (i, 0, 0)),
            pl.BlockSpec((RB, KNB, C), lambda i: (i, 0, 0)),
            pl.BlockSpec((RB, 1, C), lambda i: (i, 0, 0)),
            pl.BlockSpec((RB, C), lambda i: (i, 0)),
            small64, small64, small64,
            small16, vec16, vec16,
            vecC, vecC,
            vec64, vec64, vec16, vec16, vecC, vecC,
            pl.BlockSpec((C, 64), lambda i: (0, 0)),
        ],
        out_specs=pl.BlockSpec((RB, C), lambda i: (i, 0)),
        out_shape=jax.ShapeDtypeStruct((NPTS, C), jnp.float32),
    )(AG, APQ, PQc, Fe2d, b1r, g1r, bt1r, bwr, gwr, btwr,
      g2r, bt2r, s1, s1q, sw_, swq, s2, s2q, W2E)


def kernel(F_E, Q_prime, W1, b1, g1, bt1, W2, b2, g2, bt2, Ww, bw, gw, btw):
    W2F = W2[:, :C]
    W2E = W2[:, C:]
    G, PQ = _make_tables(F_E, Q_prime, W2F, b2.reshape(1, C), W1, Ww)
    idx = _knn(Q_prime)
    AG, APQ = _sc_gather(G.reshape(NPTS, C), PQ.reshape(NPTS, C),
                         idx.reshape(NROWS // 128, 128))
    AG = AG.reshape(NPTS, KNB, C)
    APQ = APQ.reshape(NPTS, KNB, C)
    PQc = PQ.reshape(NPTS, 1, C)
    b1r = b1.reshape(1, 1, 64)
    bwr = bw.reshape(1, 1, KNB)
    s1, s1q, sw_, swq = _stats1(APQ, PQc, b1r, bwr)
    g1r = g1.reshape(1, 1, 64)
    bt1r = bt1.reshape(1, 1, 64)
    s2, s2q = _stats2(AG, APQ, PQc, b1r, g1r, bt1r, s1, s1q, W2E)
    Fe2d = jnp.transpose(F_E, (0, 2, 1)).reshape(NPTS, C)
    out2d = _final(AG, APQ, PQc, Fe2d, b1r, g1r, bt1r,
                   bwr, gw.reshape(1, KNB), btw.reshape(1, KNB),
                   g2.reshape(1, C), bt2.reshape(1, C),
                   s1, s1q, sw_, swq, s2, s2q, W2E)
    return jnp.transpose(out2d.reshape(B, RN, C), (0, 2, 1))
